# baseline (speedup 1.0000x reference)
import functools
import jax
import jax.numpy as jnp
from jax import lax
from jax.experimental import pallas as pl
from jax.experimental.pallas import tpu as pltpu
from jax.experimental.pallas import tpu_sc as plsc

N = 10000
NPAD = 10240
D = 256
DH = 128
E = 160000
NC = 2
NS = 16
CH = 128
NCHUNK_SPMM = 80
GRP = 16
NGROUP = NCHUNK_SPMM // GRP
EPAD = NS * NCHUNK_SPMM * CH
NCHUNK_DEG = 40
ROWS_PER_TILE = NPAD // NS
BLK = 256
NBLK = NPAD // BLK

@functools.cache
def _sc_kernels():
    mesh = plsc.VectorSubcoreMesh(core_axis_name="c", subcore_axis_name="s",
                                  num_cores=NC, num_subcores=NS)
    deg = functools.partial(
        pl.kernel,
        out_type=jax.ShapeDtypeStruct((NC * NPAD, DH), jnp.float32),
        mesh=mesh,
        scratch_types=[
            pltpu.VMEM((NCHUNK_DEG, CH), jnp.int32),
            pltpu.VMEM((CH, DH), jnp.float32),
            pltpu.VMEM_SHARED((NPAD, DH), jnp.float32),
            pltpu.SemaphoreType.DMA,
        ],
    )(_sc_deg_body)
    spmm = functools.partial(
        pl.kernel,
        out_type=jax.ShapeDtypeStruct((NC * NPAD, DH), jnp.float32),
        mesh=mesh,
        scratch_types=[
            pltpu.VMEM((GRP, CH), jnp.int32),
            pltpu.VMEM((GRP, CH), jnp.int32),
            pltpu.VMEM((2, CH, DH), jnp.float32),
            pltpu.VMEM_SHARED((NPAD, DH), jnp.float32),
            [pltpu.SemaphoreType.DMA] * 4,
        ],
    )(_sc_spmm_body)
    return deg, spmm


def _sc_deg(src_deg):
    return _sc_kernels()[0](src_deg)


def _sc_spmm(hp2, src_spmm2, dst_spmm):
    return _sc_kernels()[1](hp2, src_spmm2, dst_spmm)


def _sc_deg_body(src_hbm, out_hbm, idx_v, buf_v, acc, sem):
    c = lax.axis_index("c")
    s = lax.axis_index("s")
    w = s * NC + c

    @pl.loop(0, CH)
    def _zr(i):
        for k in range(DH // 16):
            buf_v[i, pl.ds(k * 16, 16)] = jnp.zeros((16,), jnp.float32)

    @pl.loop(0, ROWS_PER_TILE // CH)
    def _zero(i):
        pltpu.sync_copy(buf_v, acc.at[pl.ds(s * ROWS_PER_TILE + i * CH, CH)])

    plsc.subcore_barrier()

    @pl.loop(0, CH)
    def _fill(i):
        for k in range(DH // 16):
            buf_v[i, pl.ds(k * 16, 16)] = jnp.full((16,), 1.0, jnp.float32)

    pltpu.sync_copy(src_hbm.at[w], idx_v)

    @pl.loop(0, NCHUNK_DEG)
    def _accum(j):
        pltpu.sync_copy(buf_v, acc.at[idx_v.at[j]], add=True)

    plsc.subcore_barrier()
    base = c * NPAD + s * ROWS_PER_TILE
    pltpu.sync_copy(acc.at[pl.ds(s * ROWS_PER_TILE, ROWS_PER_TILE)],
                    out_hbm.at[pl.ds(base, ROWS_PER_TILE)])


def _sc_spmm_body(hp_hbm, src_hbm, dst_hbm, out_hbm,
                  src_v, dst_v, rows_v, acc, sems):
    c = lax.axis_index("c")
    s = lax.axis_index("s")

    @pl.loop(0, CH)
    def _zr(i):
        for k in range(DH // 16):
            rows_v[0, i, pl.ds(k * 16, 16)] = jnp.zeros((16,), jnp.float32)

    @pl.loop(0, ROWS_PER_TILE // CH)
    def _zero(i):
        pltpu.sync_copy(rows_v.at[0],
                        acc.at[pl.ds(s * ROWS_PER_TILE + i * CH, CH)])

    plsc.subcore_barrier()

    wq = (c * NS + s) * NGROUP
    sq = s * NGROUP

    @pl.loop(0, NGROUP)
    def _group(g):
        pltpu.sync_copy(src_hbm.at[wq + g], src_v)
        pltpu.sync_copy(dst_hbm.at[sq + g], dst_v)
        for b in range(2):
            pltpu.async_copy(hp_hbm.at[src_v.at[b]], rows_v.at[b], sems[b])
        for b in range(GRP):
            rb = b & 1
            pltpu.make_async_copy(hp_hbm.at[src_v.at[b]], rows_v.at[rb],
                                  sems[rb]).wait()
            pltpu.async_copy(rows_v.at[rb], acc.at[dst_v.at[b]],
                             sems[2 + rb], add=True)
            if b + 2 < GRP:
                pltpu.make_async_copy(rows_v.at[rb], acc.at[dst_v.at[b]],
                                      sems[2 + rb]).wait()
                pltpu.async_copy(hp_hbm.at[src_v.at[b + 2]], rows_v.at[rb],
                                 sems[rb])
        for rb in range(2):
            b = GRP - 2 + rb
            pltpu.make_async_copy(rows_v.at[rb], acc.at[dst_v.at[b]],
                                  sems[2 + rb]).wait()

    plsc.subcore_barrier()
    base = c * NPAD + s * ROWS_PER_TILE
    pltpu.sync_copy(acc.at[pl.ds(s * ROWS_PER_TILE, ROWS_PER_TILE)],
                    out_hbm.at[pl.ds(base, ROWS_PER_TILE)])


def _tc_dis_hpx_body(dega_ref, degb_ref, x_ref, dis_ref, hp_ref):
    deg = dega_ref[:, :16] + degb_ref[:, :16]
    dis = jnp.where(deg > 0, lax.rsqrt(jnp.where(deg > 0, deg, 1.0)), 0.0)
    dis_ref[...] = dis
    hp_ref[...] = x_ref[...] * dis[:, 0:1]


def _tc_dis_hpx(degs, x_pad):
    return pl.pallas_call(
        _tc_dis_hpx_body,
        grid=(NBLK,),
        in_specs=[
            pl.BlockSpec((BLK, DH), lambda i: (i, 0)),
            pl.BlockSpec((BLK, DH), lambda i: (NBLK + i, 0)),
            pl.BlockSpec((BLK, D), lambda i: (i, 0)),
        ],
        out_specs=[
            pl.BlockSpec((BLK, 16), lambda i: (i, 0)),
            pl.BlockSpec((BLK, D), lambda i: (i, 0)),
        ],
        out_shape=[
            jax.ShapeDtypeStruct((NPAD, 16), jnp.float32),
            jax.ShapeDtypeStruct((NPAD, D), jnp.float32),
        ],
    )(degs, degs, x_pad)


def _tc_mma_body(h_ref, w0_ref, b_ref, a_ref):
    a_ref[...] = jnp.dot(h_ref[...], w0_ref[...],
                         preferred_element_type=jnp.float32) + b_ref[...]


def _tc_mma(h, w0, b):
    return pl.pallas_call(
        _tc_mma_body,
        grid=(NBLK,),
        in_specs=[
            pl.BlockSpec((BLK, D), lambda i: (i, 0)),
            pl.BlockSpec((D, D), lambda i: (0, 0)),
            pl.BlockSpec((1, D), lambda i: (0, 0)),
        ],
        out_specs=pl.BlockSpec((BLK, D), lambda i: (i, 0)),
        out_shape=jax.ShapeDtypeStruct((NPAD, D), jnp.float32),
    )(h, w0, b)


def _tc_mm_body(a_ref, ta_ref, tb_ref, dis_ref, w1a_ref, w1b_ref,
                r_ref, stats_ref, sacc):
    i = pl.program_id(0)
    nd = -dis_ref[:, 0:1]
    pre = a_ref[...]
    pre += jnp.dot(ta_ref[...] * nd, w1a_ref[...],
                   preferred_element_type=jnp.float32)
    pre += jnp.dot(tb_ref[...] * nd, w1b_ref[...],
                   preferred_element_type=jnp.float32)
    r = jnp.maximum(pre, 0.0)
    rows = i * BLK + lax.broadcasted_iota(jnp.int32, (BLK, 1), 0)
    r = jnp.where(rows < N, r, 0.0)
    r_ref[...] = r

    @pl.when(i == 0)
    def _():
        sacc[...] = jnp.zeros_like(sacc)

    sacc[0:1, :] += jnp.sum(r, axis=0, keepdims=True)
    sacc[1:2, :] += jnp.sum(r * r, axis=0, keepdims=True)

    @pl.when(i == NBLK - 1)
    def _():
        stats_ref[...] = sacc[...]


def _tc_mm(a, tx, dis16, w1a, w1b):
    return pl.pallas_call(
        _tc_mm_body,
        grid=(NBLK,),
        in_specs=[
            pl.BlockSpec((BLK, D), lambda i: (i, 0)),
            pl.BlockSpec((BLK, DH), lambda i: (i, 0)),
            pl.BlockSpec((BLK, DH), lambda i: (NBLK + i, 0)),
            pl.BlockSpec((BLK, 16), lambda i: (i, 0)),
            pl.BlockSpec((DH, D), lambda i: (0, 0)),
            pl.BlockSpec((DH, D), lambda i: (0, 0)),
        ],
        out_specs=[
            pl.BlockSpec((BLK, D), lambda i: (i, 0)),
            pl.BlockSpec((8, D), lambda i: (0, 0)),
        ],
        out_shape=[
            jax.ShapeDtypeStruct((NPAD, D), jnp.float32),
            jax.ShapeDtypeStruct((8, D), jnp.float32),
        ],
        scratch_shapes=[pltpu.VMEM((8, D), jnp.float32)],
        compiler_params=pltpu.CompilerParams(
            dimension_semantics=("arbitrary",)),
    )(a, tx, tx, dis16, w1a, w1b)


def _tc_fin_body(has_res, emit_hp, *refs):
    if has_res:
        (r_ref, stats_ref, g_ref, be_ref, dis_ref, res_ref), outs = \
            refs[:6], refs[6:]
    else:
        (r_ref, stats_ref, g_ref, be_ref, dis_ref), outs = refs[:5], refs[5:]
    i = pl.program_id(0)
    inv_n = 1.0 / N
    m = stats_ref[0:1, :] * inv_n
    v = stats_ref[1:2, :] * inv_n - m * m
    scale = lax.rsqrt(v + 1e-5) * g_ref[...]
    h = (r_ref[...] - m) * scale + be_ref[...]
    if has_res:
        h += res_ref[...]
    outs[0][...] = h
    if emit_hp:
        rows = i * BLK + lax.broadcasted_iota(jnp.int32, (BLK, 1), 0)
        outs[1][...] = jnp.where(rows < N, h * dis_ref[:, 0:1], 0.0)


def _tc_fin(r, stats, g, be, dis16, res=None, emit_hp=True):
    in_specs = [
        pl.BlockSpec((BLK, D), lambda i: (i, 0)),
        pl.BlockSpec((8, D), lambda i: (0, 0)),
        pl.BlockSpec((1, D), lambda i: (0, 0)),
        pl.BlockSpec((1, D), lambda i: (0, 0)),
        pl.BlockSpec((BLK, 16), lambda i: (i, 0)),
    ]
    args = [r, stats, g, be, dis16]
    if res is not None:
        in_specs.append(pl.BlockSpec((BLK, D), lambda i: (i, 0)))
        args.append(res)
    out_specs = [pl.BlockSpec((BLK, D), lambda i: (i, 0))]
    out_shape = [jax.ShapeDtypeStruct((NPAD, D), jnp.float32)]
    if emit_hp:
        out_specs.append(pl.BlockSpec((BLK, D), lambda i: (i, 0)))
        out_shape.append(jax.ShapeDtypeStruct((NPAD, D), jnp.float32))
    return pl.pallas_call(
        functools.partial(_tc_fin_body, res is not None, emit_hp),
        grid=(NBLK,),
        in_specs=in_specs,
        out_specs=out_specs,
        out_shape=out_shape,
    )(*args)


def _tc_heads_body(h_ref, wr1_ref, br1_ref, gr_ref, ber_ref, wr2_ref, br2_ref,
                   wp1_ref, bp1_ref, gp_ref, bep_ref, wp2_ref, bp2_ref,
                   out_ref):
    h = h_ref[...]

    t = jnp.dot(h, wr1_ref[...], preferred_element_type=jnp.float32) \
        + br1_ref[...]
    m = jnp.mean(t, axis=1, keepdims=True)
    v = jnp.mean(t * t, axis=1, keepdims=True) - m * m
    t = (t - m) * lax.rsqrt(v + 1e-5) * gr_ref[...] + ber_ref[...]
    t = jnp.maximum(t, 0.0)
    rad = jnp.dot(t, wr2_ref[...], preferred_element_type=jnp.float32) \
        + br2_ref[...]
    x = rad[:, 0:1]
    radius = jnp.maximum(x, 0.0) + jnp.log1p(jnp.exp(-jnp.abs(x)))

    u = jnp.dot(h, wp1_ref[...], preferred_element_type=jnp.float32) \
        + bp1_ref[...]
    m = jnp.mean(u, axis=1, keepdims=True)
    v = jnp.mean(u * u, axis=1, keepdims=True) - m * m
    u = (u - m) * lax.rsqrt(v + 1e-5) * gp_ref[...] + bep_ref[...]
    u = jnp.maximum(u, 0.0)
    pos = jnp.dot(u, wp2_ref[...], preferred_element_type=jnp.float32) \
        + bp2_ref[...]
    p0 = pos[:, 0:1]
    p1 = pos[:, 1:2]
    nrm = jnp.maximum(jnp.sqrt(p0 * p0 + p1 * p1), 1e-12)
    out_ref[...] = pos * (radius / nrm)


def _tc_heads(h3, wr1, br1, gr, ber, wr2p, br2p, wp1, bp1, gp, bep, wp2p,
              bp2p):
    return pl.pallas_call(
        _tc_heads_body,
        grid=(NBLK,),
        in_specs=[
            pl.BlockSpec((BLK, D), lambda i: (i, 0)),
            pl.BlockSpec((D, DH), lambda i: (0, 0)),
            pl.BlockSpec((1, DH), lambda i: (0, 0)),
            pl.BlockSpec((1, DH), lambda i: (0, 0)),
            pl.BlockSpec((1, DH), lambda i: (0, 0)),
            pl.BlockSpec((DH, DH), lambda i: (0, 0)),
            pl.BlockSpec((1, DH), lambda i: (0, 0)),
            pl.BlockSpec((D, D), lambda i: (0, 0)),
            pl.BlockSpec((1, D), lambda i: (0, 0)),
            pl.BlockSpec((1, D), lambda i: (0, 0)),
            pl.BlockSpec((1, D), lambda i: (0, 0)),
            pl.BlockSpec((D, DH), lambda i: (0, 0)),
            pl.BlockSpec((1, DH), lambda i: (0, 0)),
        ],
        out_specs=pl.BlockSpec((BLK, DH), lambda i: (i, 0)),
        out_shape=jax.ShapeDtypeStruct((NPAD, DH), jnp.float32),
    )(h3, wr1, br1, gr, ber, wr2p, br2p, wp1, bp1, gp, bep, wp2p, bp2p)


def kernel(x, edge_index, w0_1, w1_1, b_1, g1, be1, w0_2, w1_2, b_2, g2, be2,
           w0_3, w1_3, b_3, g3, be3, wp1, bp1, gp, bep, wp2, bp2, wr1, br1,
           gr, ber, wr2, br2):
    f32 = jnp.float32
    x_pad = jnp.zeros((NPAD, D), f32).at[:N].set(x)
    src = edge_index[0]
    dst = edge_index[1]
    pad = jnp.full((EPAD - E,), N, jnp.int32)
    srcp = jnp.concatenate([src, pad])
    dstp = jnp.concatenate([dst, pad])
    src2 = 2 * srcp
    src_spmm2 = jnp.stack([src2, src2 + 1]).reshape(NC * NS * NGROUP, GRP, CH)
    dst_spmm = dstp.reshape(NS * NGROUP, GRP, CH)
    src_deg = srcp.reshape(NC * NS, NCHUNK_DEG, CH)

    def row(b):
        return b.reshape(1, -1)

    degs = _sc_deg(src_deg)
    dis16, hp = _tc_dis_hpx(degs, x_pad)

    layers = [
        (w0_1, w1_1, b_1, g1, be1),
        (w0_2, w1_2, b_2, g2, be2),
        (w0_3, w1_3, b_3, g3, be3),
    ]
    h = x_pad
    res = None
    for li, (w0, w1, b, g, be) in enumerate(layers):
        tx = _sc_spmm(hp.reshape(NC * NPAD, DH), src_spmm2, dst_spmm)
        a = _tc_mma(h, w0, row(b))
        r, stats = _tc_mm(a, tx, dis16, w1[:DH], w1[DH:])
        emit_hp = li < 2
        outs = _tc_fin(r, stats, row(g), row(be), dis16, res=res,
                       emit_hp=emit_hp)
        if emit_hp:
            h_new, hp = outs
        else:
            (h_new,) = outs
        res = h_new
        h = h_new

    wr2p = jnp.pad(wr2, ((0, 0), (0, DH - 1)))
    br2p = jnp.pad(br2, (0, DH - 1)).reshape(1, DH)
    wp2p = jnp.pad(wp2, ((0, 0), (0, DH - 2)))
    bp2p = jnp.pad(bp2, (0, DH - 2)).reshape(1, DH)
    coords = _tc_heads(h, wr1, row(br1), row(gr), row(ber), wr2p, br2p,
                       wp1, row(bp1), row(gp), row(bep), wp2p, bp2p)
    return coords[:N, :2]


# speedup vs baseline: 1.0345x; 1.0345x over previous
import functools
import jax
import jax.numpy as jnp
from jax import lax
from jax.experimental import pallas as pl
from jax.experimental.pallas import tpu as pltpu
from jax.experimental.pallas import tpu_sc as plsc

N = 10000
NPAD = 10240
D = 256
DH = 128
E = 160000
NC = 2
NS = 16
CH = 128
CHS = 80
NCHUNK_SPMM = 128
GRP = 16
NGROUP = NCHUNK_SPMM // GRP
EPAD = NS * NCHUNK_SPMM * CHS
NCHUNK_DEG = 40
ROWS_PER_TILE = NPAD // NS
BLK = 256
NBLK = NPAD // BLK

@functools.cache
def _sc_kernels():
    mesh = plsc.VectorSubcoreMesh(core_axis_name="c", subcore_axis_name="s",
                                  num_cores=NC, num_subcores=NS)
    deg = functools.partial(
        pl.kernel,
        out_type=jax.ShapeDtypeStruct((NC * NPAD, DH), jnp.float32),
        mesh=mesh,
        scratch_types=[
            pltpu.VMEM((NCHUNK_DEG, CH), jnp.int32),
            pltpu.VMEM((CH, DH), jnp.float32),
            pltpu.VMEM_SHARED((NPAD, DH), jnp.float32),
            pltpu.SemaphoreType.DMA,
        ],
    )(_sc_deg_body)
    spmm = functools.partial(
        pl.kernel,
        out_type=jax.ShapeDtypeStruct((NC * NPAD, DH), jnp.float32),
        mesh=mesh,
        scratch_types=[
            pltpu.VMEM((GRP, CHS), jnp.int32),
            pltpu.VMEM((GRP, CHS), jnp.int32),
            pltpu.VMEM((3, CHS, DH), jnp.float32),
            pltpu.VMEM_SHARED((NPAD, DH), jnp.float32),
            [pltpu.SemaphoreType.DMA] * 6,
        ],
    )(_sc_spmm_body)
    return deg, spmm


def _sc_deg(src_deg):
    return _sc_kernels()[0](src_deg)


def _sc_spmm(hp2, src_spmm2, dst_spmm):
    return _sc_kernels()[1](hp2, src_spmm2, dst_spmm)


def _sc_deg_body(src_hbm, out_hbm, idx_v, buf_v, acc, sem):
    c = lax.axis_index("c")
    s = lax.axis_index("s")
    w = s * NC + c

    @pl.loop(0, CH)
    def _zr(i):
        for k in range(DH // 16):
            buf_v[i, pl.ds(k * 16, 16)] = jnp.zeros((16,), jnp.float32)

    @pl.loop(0, ROWS_PER_TILE // CH)
    def _zero(i):
        pltpu.sync_copy(buf_v, acc.at[pl.ds(s * ROWS_PER_TILE + i * CH, CH)])

    plsc.subcore_barrier()

    @pl.loop(0, CH)
    def _fill(i):
        for k in range(DH // 16):
            buf_v[i, pl.ds(k * 16, 16)] = jnp.full((16,), 1.0, jnp.float32)

    pltpu.sync_copy(src_hbm.at[w], idx_v)

    @pl.loop(0, NCHUNK_DEG)
    def _accum(j):
        pltpu.sync_copy(buf_v, acc.at[idx_v.at[j]], add=True)

    plsc.subcore_barrier()
    base = c * NPAD + s * ROWS_PER_TILE
    pltpu.sync_copy(acc.at[pl.ds(s * ROWS_PER_TILE, ROWS_PER_TILE)],
                    out_hbm.at[pl.ds(base, ROWS_PER_TILE)])


def _sc_spmm_body(hp_hbm, src_hbm, dst_hbm, out_hbm,
                  src_v, dst_v, rows_v, acc, sems):
    c = lax.axis_index("c")
    s = lax.axis_index("s")

    @pl.loop(0, CHS)
    def _zr(i):
        for k in range(DH // 16):
            rows_v[0, i, pl.ds(k * 16, 16)] = jnp.zeros((16,), jnp.float32)

    @pl.loop(0, ROWS_PER_TILE // CHS)
    def _zero(i):
        pltpu.sync_copy(rows_v.at[0],
                        acc.at[pl.ds(s * ROWS_PER_TILE + i * CHS, CHS)])

    plsc.subcore_barrier()

    wq = (c * NS + s) * NGROUP
    sq = s * NGROUP

    @pl.loop(0, NGROUP)
    def _group(g):
        pltpu.sync_copy(src_hbm.at[wq + g], src_v)
        pltpu.sync_copy(dst_hbm.at[sq + g], dst_v)
        for b in range(3):
            pltpu.async_copy(hp_hbm.at[src_v.at[b]], rows_v.at[b], sems[b])
        for b in range(GRP):
            rb = b % 3
            pltpu.make_async_copy(hp_hbm.at[src_v.at[b]], rows_v.at[rb],
                                  sems[rb]).wait()
            pltpu.async_copy(rows_v.at[rb], acc.at[dst_v.at[b]],
                             sems[3 + rb], add=True)
            if b >= 1 and b + 2 < GRP:
                pb = (b - 1) % 3
                pltpu.make_async_copy(rows_v.at[pb], acc.at[dst_v.at[b - 1]],
                                      sems[3 + pb]).wait()
                pltpu.async_copy(hp_hbm.at[src_v.at[b + 2]], rows_v.at[pb],
                                 sems[pb])
        for k in range(3):
            b = GRP - 3 + k
            pltpu.make_async_copy(rows_v.at[b % 3], acc.at[dst_v.at[b]],
                                  sems[3 + (b % 3)]).wait()

    plsc.subcore_barrier()
    base = c * NPAD + s * ROWS_PER_TILE
    pltpu.sync_copy(acc.at[pl.ds(s * ROWS_PER_TILE, ROWS_PER_TILE)],
                    out_hbm.at[pl.ds(base, ROWS_PER_TILE)])


def _tc_dis_hpx_body(dega_ref, degb_ref, x_ref, dis_ref, hp_ref):
    deg = dega_ref[:, :16] + degb_ref[:, :16]
    dis = jnp.where(deg > 0, lax.rsqrt(jnp.where(deg > 0, deg, 1.0)), 0.0)
    dis_ref[...] = dis
    hp_ref[...] = x_ref[...] * dis[:, 0:1]


def _tc_dis_hpx(degs, x_pad):
    return pl.pallas_call(
        _tc_dis_hpx_body,
        grid=(NBLK,),
        in_specs=[
            pl.BlockSpec((BLK, DH), lambda i: (i, 0)),
            pl.BlockSpec((BLK, DH), lambda i: (NBLK + i, 0)),
            pl.BlockSpec((BLK, D), lambda i: (i, 0)),
        ],
        out_specs=[
            pl.BlockSpec((BLK, 16), lambda i: (i, 0)),
            pl.BlockSpec((BLK, D), lambda i: (i, 0)),
        ],
        out_shape=[
            jax.ShapeDtypeStruct((NPAD, 16), jnp.float32),
            jax.ShapeDtypeStruct((NPAD, D), jnp.float32),
        ],
    )(degs, degs, x_pad)


def _tc_mma_body(h_ref, w0_ref, b_ref, a_ref):
    a_ref[...] = jnp.dot(h_ref[...], w0_ref[...],
                         preferred_element_type=jnp.float32) + b_ref[...]


def _tc_mma(h, w0, b):
    return pl.pallas_call(
        _tc_mma_body,
        grid=(NBLK,),
        in_specs=[
            pl.BlockSpec((BLK, D), lambda i: (i, 0)),
            pl.BlockSpec((D, D), lambda i: (0, 0)),
            pl.BlockSpec((1, D), lambda i: (0, 0)),
        ],
        out_specs=pl.BlockSpec((BLK, D), lambda i: (i, 0)),
        out_shape=jax.ShapeDtypeStruct((NPAD, D), jnp.float32),
    )(h, w0, b)


def _tc_mm_body(a_ref, ta_ref, tb_ref, dis_ref, w1a_ref, w1b_ref,
                r_ref, stats_ref, sacc):
    i = pl.program_id(0)
    nd = -dis_ref[:, 0:1]
    pre = a_ref[...]
    pre += jnp.dot(ta_ref[...] * nd, w1a_ref[...],
                   preferred_element_type=jnp.float32)
    pre += jnp.dot(tb_ref[...] * nd, w1b_ref[...],
                   preferred_element_type=jnp.float32)
    r = jnp.maximum(pre, 0.0)
    rows = i * BLK + lax.broadcasted_iota(jnp.int32, (BLK, 1), 0)
    r = jnp.where(rows < N, r, 0.0)
    r_ref[...] = r

    @pl.when(i == 0)
    def _():
        sacc[...] = jnp.zeros_like(sacc)

    sacc[0:1, :] += jnp.sum(r, axis=0, keepdims=True)
    sacc[1:2, :] += jnp.sum(r * r, axis=0, keepdims=True)

    @pl.when(i == NBLK - 1)
    def _():
        stats_ref[...] = sacc[...]


def _tc_mm(a, tx, dis16, w1a, w1b):
    return pl.pallas_call(
        _tc_mm_body,
        grid=(NBLK,),
        in_specs=[
            pl.BlockSpec((BLK, D), lambda i: (i, 0)),
            pl.BlockSpec((BLK, DH), lambda i: (i, 0)),
            pl.BlockSpec((BLK, DH), lambda i: (NBLK + i, 0)),
            pl.BlockSpec((BLK, 16), lambda i: (i, 0)),
            pl.BlockSpec((DH, D), lambda i: (0, 0)),
            pl.BlockSpec((DH, D), lambda i: (0, 0)),
        ],
        out_specs=[
            pl.BlockSpec((BLK, D), lambda i: (i, 0)),
            pl.BlockSpec((8, D), lambda i: (0, 0)),
        ],
        out_shape=[
            jax.ShapeDtypeStruct((NPAD, D), jnp.float32),
            jax.ShapeDtypeStruct((8, D), jnp.float32),
        ],
        scratch_shapes=[pltpu.VMEM((8, D), jnp.float32)],
        compiler_params=pltpu.CompilerParams(
            dimension_semantics=("arbitrary",)),
    )(a, tx, tx, dis16, w1a, w1b)


def _tc_fin_body(has_res, emit_hp, *refs):
    if has_res:
        (r_ref, stats_ref, g_ref, be_ref, dis_ref, res_ref), outs = \
            refs[:6], refs[6:]
    else:
        (r_ref, stats_ref, g_ref, be_ref, dis_ref), outs = refs[:5], refs[5:]
    i = pl.program_id(0)
    inv_n = 1.0 / N
    m = stats_ref[0:1, :] * inv_n
    v = stats_ref[1:2, :] * inv_n - m * m
    scale = lax.rsqrt(v + 1e-5) * g_ref[...]
    h = (r_ref[...] - m) * scale + be_ref[...]
    if has_res:
        h += res_ref[...]
    outs[0][...] = h
    if emit_hp:
        rows = i * BLK + lax.broadcasted_iota(jnp.int32, (BLK, 1), 0)
        outs[1][...] = jnp.where(rows < N, h * dis_ref[:, 0:1], 0.0)


def _tc_fin(r, stats, g, be, dis16, res=None, emit_hp=True):
    in_specs = [
        pl.BlockSpec((BLK, D), lambda i: (i, 0)),
        pl.BlockSpec((8, D), lambda i: (0, 0)),
        pl.BlockSpec((1, D), lambda i: (0, 0)),
        pl.BlockSpec((1, D), lambda i: (0, 0)),
        pl.BlockSpec((BLK, 16), lambda i: (i, 0)),
    ]
    args = [r, stats, g, be, dis16]
    if res is not None:
        in_specs.append(pl.BlockSpec((BLK, D), lambda i: (i, 0)))
        args.append(res)
    out_specs = [pl.BlockSpec((BLK, D), lambda i: (i, 0))]
    out_shape = [jax.ShapeDtypeStruct((NPAD, D), jnp.float32)]
    if emit_hp:
        out_specs.append(pl.BlockSpec((BLK, D), lambda i: (i, 0)))
        out_shape.append(jax.ShapeDtypeStruct((NPAD, D), jnp.float32))
    return pl.pallas_call(
        functools.partial(_tc_fin_body, res is not None, emit_hp),
        grid=(NBLK,),
        in_specs=in_specs,
        out_specs=out_specs,
        out_shape=out_shape,
    )(*args)


def _tc_heads_body(h_ref, wr1_ref, br1_ref, gr_ref, ber_ref, wr2_ref, br2_ref,
                   wp1_ref, bp1_ref, gp_ref, bep_ref, wp2_ref, bp2_ref,
                   out_ref):
    h = h_ref[...]

    t = jnp.dot(h, wr1_ref[...], preferred_element_type=jnp.float32) \
        + br1_ref[...]
    m = jnp.mean(t, axis=1, keepdims=True)
    v = jnp.mean(t * t, axis=1, keepdims=True) - m * m
    t = (t - m) * lax.rsqrt(v + 1e-5) * gr_ref[...] + ber_ref[...]
    t = jnp.maximum(t, 0.0)
    rad = jnp.dot(t, wr2_ref[...], preferred_element_type=jnp.float32) \
        + br2_ref[...]
    x = rad[:, 0:1]
    radius = jnp.maximum(x, 0.0) + jnp.log1p(jnp.exp(-jnp.abs(x)))

    u = jnp.dot(h, wp1_ref[...], preferred_element_type=jnp.float32) \
        + bp1_ref[...]
    m = jnp.mean(u, axis=1, keepdims=True)
    v = jnp.mean(u * u, axis=1, keepdims=True) - m * m
    u = (u - m) * lax.rsqrt(v + 1e-5) * gp_ref[...] + bep_ref[...]
    u = jnp.maximum(u, 0.0)
    pos = jnp.dot(u, wp2_ref[...], preferred_element_type=jnp.float32) \
        + bp2_ref[...]
    p0 = pos[:, 0:1]
    p1 = pos[:, 1:2]
    nrm = jnp.maximum(jnp.sqrt(p0 * p0 + p1 * p1), 1e-12)
    out_ref[...] = pos * (radius / nrm)


def _tc_heads(h3, wr1, br1, gr, ber, wr2p, br2p, wp1, bp1, gp, bep, wp2p,
              bp2p):
    return pl.pallas_call(
        _tc_heads_body,
        grid=(NBLK,),
        in_specs=[
            pl.BlockSpec((BLK, D), lambda i: (i, 0)),
            pl.BlockSpec((D, DH), lambda i: (0, 0)),
            pl.BlockSpec((1, DH), lambda i: (0, 0)),
            pl.BlockSpec((1, DH), lambda i: (0, 0)),
            pl.BlockSpec((1, DH), lambda i: (0, 0)),
            pl.BlockSpec((DH, DH), lambda i: (0, 0)),
            pl.BlockSpec((1, DH), lambda i: (0, 0)),
            pl.BlockSpec((D, D), lambda i: (0, 0)),
            pl.BlockSpec((1, D), lambda i: (0, 0)),
            pl.BlockSpec((1, D), lambda i: (0, 0)),
            pl.BlockSpec((1, D), lambda i: (0, 0)),
            pl.BlockSpec((D, DH), lambda i: (0, 0)),
            pl.BlockSpec((1, DH), lambda i: (0, 0)),
        ],
        out_specs=pl.BlockSpec((BLK, DH), lambda i: (i, 0)),
        out_shape=jax.ShapeDtypeStruct((NPAD, DH), jnp.float32),
    )(h3, wr1, br1, gr, ber, wr2p, br2p, wp1, bp1, gp, bep, wp2p, bp2p)


def kernel(x, edge_index, w0_1, w1_1, b_1, g1, be1, w0_2, w1_2, b_2, g2, be2,
           w0_3, w1_3, b_3, g3, be3, wp1, bp1, gp, bep, wp2, bp2, wr1, br1,
           gr, ber, wr2, br2):
    f32 = jnp.float32
    x_pad = jnp.zeros((NPAD, D), f32).at[:N].set(x)
    src = edge_index[0]
    dst = edge_index[1]
    pad = jnp.full((EPAD - E,), N, jnp.int32)
    srcp = jnp.concatenate([src, pad])
    dstp = jnp.concatenate([dst, pad])
    src2 = 2 * srcp
    src_spmm2 = jnp.stack([src2, src2 + 1]).reshape(NC * NS * NGROUP, GRP, CHS)
    dst_spmm = dstp.reshape(NS * NGROUP, GRP, CHS)
    src_deg = srcp.reshape(NC * NS, NCHUNK_DEG, CH)

    def row(b):
        return b.reshape(1, -1)

    degs = _sc_deg(src_deg)
    dis16, hp = _tc_dis_hpx(degs, x_pad)

    layers = [
        (w0_1, w1_1, b_1, g1, be1),
        (w0_2, w1_2, b_2, g2, be2),
        (w0_3, w1_3, b_3, g3, be3),
    ]
    h = x_pad
    res = None
    for li, (w0, w1, b, g, be) in enumerate(layers):
        tx = _sc_spmm(hp.reshape(NC * NPAD, DH), src_spmm2, dst_spmm)
        a = _tc_mma(h, w0, row(b))
        r, stats = _tc_mm(a, tx, dis16, w1[:DH], w1[DH:])
        emit_hp = li < 2
        outs = _tc_fin(r, stats, row(g), row(be), dis16, res=res,
                       emit_hp=emit_hp)
        if emit_hp:
            h_new, hp = outs
        else:
            (h_new,) = outs
        res = h_new
        h = h_new

    wr2p = jnp.pad(wr2, ((0, 0), (0, DH - 1)))
    br2p = jnp.pad(br2, (0, DH - 1)).reshape(1, DH)
    wp2p = jnp.pad(wp2, ((0, 0), (0, DH - 2)))
    bp2p = jnp.pad(bp2, (0, DH - 2)).reshape(1, DH)
    coords = _tc_heads(h, wr1, row(br1), row(gr), row(ber), wr2p, br2p,
                       wp1, row(bp1), row(gp), row(bep), wp2p, bp2p)
    return coords[:N, :2]


# speedup vs baseline: 1.0968x; 1.0602x over previous
import functools
import jax
import jax.numpy as jnp
from jax import lax
from jax.experimental import pallas as pl
from jax.experimental.pallas import tpu as pltpu
from jax.experimental.pallas import tpu_sc as plsc

N = 10000
NPAD = 10240
D = 256
DH = 128
E = 160000
NC = 2
NS = 16
CH = 128
CHS = 80
NCHUNK_SPMM = 128
GRP = 16
NGROUP = NCHUNK_SPMM // GRP
EPAD = NS * NCHUNK_SPMM * CHS
NCHUNK_DEG = 40
ROWS_PER_TILE = NPAD // NS
BLK = 256
NBLK = NPAD // BLK

@functools.cache
def _sc_kernels():
    mesh = plsc.VectorSubcoreMesh(core_axis_name="c", subcore_axis_name="s",
                                  num_cores=NC, num_subcores=NS)
    deg = functools.partial(
        pl.kernel,
        out_type=jax.ShapeDtypeStruct((NC * NPAD, DH), jnp.float32),
        mesh=mesh,
        scratch_types=[
            pltpu.VMEM((NCHUNK_DEG, CH), jnp.int32),
            pltpu.VMEM((CH, DH), jnp.float32),
            pltpu.VMEM_SHARED((NPAD, DH), jnp.float32),
            pltpu.SemaphoreType.DMA,
        ],
    )(_sc_deg_body)
    spmm = functools.partial(
        pl.kernel,
        out_type=jax.ShapeDtypeStruct((NC * NPAD, DH), jnp.float32),
        mesh=mesh,
        scratch_types=[
            pltpu.VMEM((GRP, CHS), jnp.int32),
            pltpu.VMEM((GRP, CHS), jnp.int32),
            pltpu.VMEM((3, CHS, DH), jnp.float32),
            pltpu.VMEM_SHARED((NPAD, DH), jnp.float32),
            [pltpu.SemaphoreType.DMA] * 6,
        ],
    )(_sc_spmm_body)
    return deg, spmm


def _sc_deg(src_deg):
    return _sc_kernels()[0](src_deg)


def _sc_spmm(hp2, src_spmm2, dst_spmm):
    return _sc_kernels()[1](hp2, src_spmm2, dst_spmm)


def _sc_deg_body(src_hbm, out_hbm, idx_v, buf_v, acc, sem):
    c = lax.axis_index("c")
    s = lax.axis_index("s")
    w = s * NC + c

    @pl.loop(0, CH)
    def _zr(i):
        for k in range(DH // 16):
            buf_v[i, pl.ds(k * 16, 16)] = jnp.zeros((16,), jnp.float32)

    @pl.loop(0, ROWS_PER_TILE // CH)
    def _zero(i):
        pltpu.sync_copy(buf_v, acc.at[pl.ds(s * ROWS_PER_TILE + i * CH, CH)])

    plsc.subcore_barrier()

    @pl.loop(0, CH)
    def _fill(i):
        for k in range(DH // 16):
            buf_v[i, pl.ds(k * 16, 16)] = jnp.full((16,), 1.0, jnp.float32)

    pltpu.sync_copy(src_hbm.at[w], idx_v)

    @pl.loop(0, NCHUNK_DEG)
    def _accum(j):
        pltpu.async_copy(buf_v, acc.at[idx_v.at[j]], sem, add=True)

    @pl.loop(0, NCHUNK_DEG)
    def _drain(j):
        pltpu.make_async_copy(buf_v, acc.at[idx_v.at[j]], sem).wait()

    plsc.subcore_barrier()
    base = c * NPAD + s * ROWS_PER_TILE
    pltpu.sync_copy(acc.at[pl.ds(s * ROWS_PER_TILE, ROWS_PER_TILE)],
                    out_hbm.at[pl.ds(base, ROWS_PER_TILE)])


def _sc_spmm_body(hp_hbm, src_hbm, dst_hbm, out_hbm,
                  src_v, dst_v, rows_v, acc, sems):
    c = lax.axis_index("c")
    s = lax.axis_index("s")

    @pl.loop(0, CHS)
    def _zr(i):
        for k in range(DH // 16):
            rows_v[0, i, pl.ds(k * 16, 16)] = jnp.zeros((16,), jnp.float32)

    @pl.loop(0, ROWS_PER_TILE // CHS)
    def _zero(i):
        pltpu.sync_copy(rows_v.at[0],
                        acc.at[pl.ds(s * ROWS_PER_TILE + i * CHS, CHS)])

    plsc.subcore_barrier()

    wq = (c * NS + s) * NGROUP
    sq = s * NGROUP

    @pl.loop(0, NGROUP)
    def _group(g):
        pltpu.sync_copy(src_hbm.at[wq + g], src_v)
        pltpu.sync_copy(dst_hbm.at[sq + g], dst_v)
        for b in range(3):
            pltpu.async_copy(hp_hbm.at[src_v.at[b]], rows_v.at[b], sems[b])
        for b in range(GRP):
            rb = b % 3
            pltpu.make_async_copy(hp_hbm.at[src_v.at[b]], rows_v.at[rb],
                                  sems[rb]).wait()
            pltpu.async_copy(rows_v.at[rb], acc.at[dst_v.at[b]],
                             sems[3 + rb], add=True)
            if b >= 1 and b + 2 < GRP:
                pb = (b - 1) % 3
                pltpu.make_async_copy(rows_v.at[pb], acc.at[dst_v.at[b - 1]],
                                      sems[3 + pb]).wait()
                pltpu.async_copy(hp_hbm.at[src_v.at[b + 2]], rows_v.at[pb],
                                 sems[pb])
        for k in range(3):
            b = GRP - 3 + k
            pltpu.make_async_copy(rows_v.at[b % 3], acc.at[dst_v.at[b]],
                                  sems[3 + (b % 3)]).wait()

    plsc.subcore_barrier()
    base = c * NPAD + s * ROWS_PER_TILE
    pltpu.sync_copy(acc.at[pl.ds(s * ROWS_PER_TILE, ROWS_PER_TILE)],
                    out_hbm.at[pl.ds(base, ROWS_PER_TILE)])


def _tc_dis_hpx_body(dega_ref, degb_ref, x_ref, dis_ref, hp_ref):
    deg = dega_ref[:, :16] + degb_ref[:, :16]
    dis = jnp.where(deg > 0, lax.rsqrt(jnp.where(deg > 0, deg, 1.0)), 0.0)
    dis_ref[...] = dis
    hp_ref[...] = (x_ref[...] * dis[:, 0:1]).reshape(2 * BLK, DH)


def _tc_dis_hpx(degs, x_pad):
    return pl.pallas_call(
        _tc_dis_hpx_body,
        grid=(NBLK,),
        in_specs=[
            pl.BlockSpec((BLK, DH), lambda i: (i, 0)),
            pl.BlockSpec((BLK, DH), lambda i: (NBLK + i, 0)),
            pl.BlockSpec((BLK, D), lambda i: (i, 0)),
        ],
        out_specs=[
            pl.BlockSpec((BLK, 16), lambda i: (i, 0)),
            pl.BlockSpec((2 * BLK, DH), lambda i: (i, 0)),
        ],
        out_shape=[
            jax.ShapeDtypeStruct((NPAD, 16), jnp.float32),
            jax.ShapeDtypeStruct((2 * NPAD, DH), jnp.float32),
        ],
    )(degs, degs, x_pad)


def _tc_mma_body(h_ref, w0_ref, b_ref, a_ref):
    a_ref[...] = jnp.dot(h_ref[...], w0_ref[...],
                         preferred_element_type=jnp.float32) + b_ref[...]


def _tc_mma(h, w0, b):
    return pl.pallas_call(
        _tc_mma_body,
        grid=(NBLK,),
        in_specs=[
            pl.BlockSpec((BLK, D), lambda i: (i, 0)),
            pl.BlockSpec((D, D), lambda i: (0, 0)),
            pl.BlockSpec((1, D), lambda i: (0, 0)),
        ],
        out_specs=pl.BlockSpec((BLK, D), lambda i: (i, 0)),
        out_shape=jax.ShapeDtypeStruct((NPAD, D), jnp.float32),
    )(h, w0, b)


def _tc_mm_body(a_ref, ta_ref, tb_ref, dis_ref, w1a_ref, w1b_ref,
                r_ref, stats_ref, sacc):
    i = pl.program_id(0)
    nd = -dis_ref[:, 0:1]
    pre = a_ref[...]
    pre += jnp.dot(ta_ref[...] * nd, w1a_ref[...],
                   preferred_element_type=jnp.float32)
    pre += jnp.dot(tb_ref[...] * nd, w1b_ref[...],
                   preferred_element_type=jnp.float32)
    r = jnp.maximum(pre, 0.0)
    rows = i * BLK + lax.broadcasted_iota(jnp.int32, (BLK, 1), 0)
    r = jnp.where(rows < N, r, 0.0)
    r_ref[...] = r

    @pl.when(i == 0)
    def _():
        sacc[...] = jnp.zeros_like(sacc)

    sacc[0:1, :] += jnp.sum(r, axis=0, keepdims=True)
    sacc[1:2, :] += jnp.sum(r * r, axis=0, keepdims=True)

    @pl.when(i == NBLK - 1)
    def _():
        stats_ref[...] = sacc[...]


def _tc_mm(a, tx, dis16, w1a, w1b):
    return pl.pallas_call(
        _tc_mm_body,
        grid=(NBLK,),
        in_specs=[
            pl.BlockSpec((BLK, D), lambda i: (i, 0)),
            pl.BlockSpec((BLK, DH), lambda i: (i, 0)),
            pl.BlockSpec((BLK, DH), lambda i: (NBLK + i, 0)),
            pl.BlockSpec((BLK, 16), lambda i: (i, 0)),
            pl.BlockSpec((DH, D), lambda i: (0, 0)),
            pl.BlockSpec((DH, D), lambda i: (0, 0)),
        ],
        out_specs=[
            pl.BlockSpec((BLK, D), lambda i: (i, 0)),
            pl.BlockSpec((8, D), lambda i: (0, 0)),
        ],
        out_shape=[
            jax.ShapeDtypeStruct((NPAD, D), jnp.float32),
            jax.ShapeDtypeStruct((8, D), jnp.float32),
        ],
        scratch_shapes=[pltpu.VMEM((8, D), jnp.float32)],
        compiler_params=pltpu.CompilerParams(
            dimension_semantics=("arbitrary",)),
    )(a, tx, tx, dis16, w1a, w1b)


def _tc_fin_body(has_res, emit_hp, *refs):
    if has_res:
        (r_ref, stats_ref, g_ref, be_ref, dis_ref, res_ref), outs = \
            refs[:6], refs[6:]
    else:
        (r_ref, stats_ref, g_ref, be_ref, dis_ref), outs = refs[:5], refs[5:]
    i = pl.program_id(0)
    inv_n = 1.0 / N
    m = stats_ref[0:1, :] * inv_n
    v = stats_ref[1:2, :] * inv_n - m * m
    scale = lax.rsqrt(v + 1e-5) * g_ref[...]
    h = (r_ref[...] - m) * scale + be_ref[...]
    if has_res:
        h += res_ref[...]
    outs[0][...] = h
    if emit_hp:
        rows = i * BLK + lax.broadcasted_iota(jnp.int32, (BLK, 1), 0)
        hp = jnp.where(rows < N, h * dis_ref[:, 0:1], 0.0)
        outs[1][...] = hp.reshape(2 * BLK, DH)


def _tc_fin(r, stats, g, be, dis16, res=None, emit_hp=True):
    in_specs = [
        pl.BlockSpec((BLK, D), lambda i: (i, 0)),
        pl.BlockSpec((8, D), lambda i: (0, 0)),
        pl.BlockSpec((1, D), lambda i: (0, 0)),
        pl.BlockSpec((1, D), lambda i: (0, 0)),
        pl.BlockSpec((BLK, 16), lambda i: (i, 0)),
    ]
    args = [r, stats, g, be, dis16]
    if res is not None:
        in_specs.append(pl.BlockSpec((BLK, D), lambda i: (i, 0)))
        args.append(res)
    out_specs = [pl.BlockSpec((BLK, D), lambda i: (i, 0))]
    out_shape = [jax.ShapeDtypeStruct((NPAD, D), jnp.float32)]
    if emit_hp:
        out_specs.append(pl.BlockSpec((2 * BLK, DH), lambda i: (i, 0)))
        out_shape.append(jax.ShapeDtypeStruct((2 * NPAD, DH), jnp.float32))
    return pl.pallas_call(
        functools.partial(_tc_fin_body, res is not None, emit_hp),
        grid=(NBLK,),
        in_specs=in_specs,
        out_specs=out_specs,
        out_shape=out_shape,
    )(*args)


def _tc_heads_body(r_ref, stats_ref, g_ref, be_ref, res_ref,
                   wr1_ref, br1_ref, gr_ref, ber_ref, wr2_ref, br2_ref,
                   wp1_ref, bp1_ref, gp_ref, bep_ref, wp2_ref, bp2_ref,
                   out_ref):
    inv_n = 1.0 / N
    m = stats_ref[0:1, :] * inv_n
    v = stats_ref[1:2, :] * inv_n - m * m
    h = (r_ref[...] - m) * (lax.rsqrt(v + 1e-5) * g_ref[...]) + be_ref[...] \
        + res_ref[...]

    t = jnp.dot(h, wr1_ref[...], preferred_element_type=jnp.float32) \
        + br1_ref[...]
    m = jnp.mean(t, axis=1, keepdims=True)
    v = jnp.mean(t * t, axis=1, keepdims=True) - m * m
    t = (t - m) * lax.rsqrt(v + 1e-5) * gr_ref[...] + ber_ref[...]
    t = jnp.maximum(t, 0.0)
    rad = jnp.dot(t, wr2_ref[...], preferred_element_type=jnp.float32) \
        + br2_ref[...]
    x = rad[:, 0:1]
    radius = jnp.maximum(x, 0.0) + jnp.log1p(jnp.exp(-jnp.abs(x)))

    u = jnp.dot(h, wp1_ref[...], preferred_element_type=jnp.float32) \
        + bp1_ref[...]
    m = jnp.mean(u, axis=1, keepdims=True)
    v = jnp.mean(u * u, axis=1, keepdims=True) - m * m
    u = (u - m) * lax.rsqrt(v + 1e-5) * gp_ref[...] + bep_ref[...]
    u = jnp.maximum(u, 0.0)
    pos = jnp.dot(u, wp2_ref[...], preferred_element_type=jnp.float32) \
        + bp2_ref[...]
    p0 = pos[:, 0:1]
    p1 = pos[:, 1:2]
    nrm = jnp.maximum(jnp.sqrt(p0 * p0 + p1 * p1), 1e-12)
    out_ref[...] = pos * (radius / nrm)


def _tc_heads(r3, stats3, g3, be3, res, wr1, br1, gr, ber, wr2p, br2p,
              wp1, bp1, gp, bep, wp2p, bp2p):
    return pl.pallas_call(
        _tc_heads_body,
        grid=(NBLK,),
        in_specs=[
            pl.BlockSpec((BLK, D), lambda i: (i, 0)),
            pl.BlockSpec((8, D), lambda i: (0, 0)),
            pl.BlockSpec((1, D), lambda i: (0, 0)),
            pl.BlockSpec((1, D), lambda i: (0, 0)),
            pl.BlockSpec((BLK, D), lambda i: (i, 0)),
            pl.BlockSpec((D, DH), lambda i: (0, 0)),
            pl.BlockSpec((1, DH), lambda i: (0, 0)),
            pl.BlockSpec((1, DH), lambda i: (0, 0)),
            pl.BlockSpec((1, DH), lambda i: (0, 0)),
            pl.BlockSpec((DH, DH), lambda i: (0, 0)),
            pl.BlockSpec((1, DH), lambda i: (0, 0)),
            pl.BlockSpec((D, D), lambda i: (0, 0)),
            pl.BlockSpec((1, D), lambda i: (0, 0)),
            pl.BlockSpec((1, D), lambda i: (0, 0)),
            pl.BlockSpec((1, D), lambda i: (0, 0)),
            pl.BlockSpec((D, DH), lambda i: (0, 0)),
            pl.BlockSpec((1, DH), lambda i: (0, 0)),
        ],
        out_specs=pl.BlockSpec((BLK, DH), lambda i: (i, 0)),
        out_shape=jax.ShapeDtypeStruct((NPAD, DH), jnp.float32),
    )(r3, stats3, g3, be3, res, wr1, br1, gr, ber, wr2p, br2p,
      wp1, bp1, gp, bep, wp2p, bp2p)


def kernel(x, edge_index, w0_1, w1_1, b_1, g1, be1, w0_2, w1_2, b_2, g2, be2,
           w0_3, w1_3, b_3, g3, be3, wp1, bp1, gp, bep, wp2, bp2, wr1, br1,
           gr, ber, wr2, br2):
    f32 = jnp.float32
    x_pad = jnp.zeros((NPAD, D), f32).at[:N].set(x)
    src = edge_index[0]
    dst = edge_index[1]
    pad = jnp.full((EPAD - E,), N, jnp.int32)
    srcp = jnp.concatenate([src, pad])
    dstp = jnp.concatenate([dst, pad])
    src2 = 2 * srcp
    src_spmm2 = jnp.stack([src2, src2 + 1]).reshape(NC * NS * NGROUP, GRP, CHS)
    dst_spmm = dstp.reshape(NS * NGROUP, GRP, CHS)
    src_deg = srcp.reshape(NC * NS, NCHUNK_DEG, CH)

    def row(b):
        return b.reshape(1, -1)

    degs = _sc_deg(src_deg)
    dis16, hp = _tc_dis_hpx(degs, x_pad)

    layers = [
        (w0_1, w1_1, b_1, g1, be1),
        (w0_2, w1_2, b_2, g2, be2),
        (w0_3, w1_3, b_3, g3, be3),
    ]
    h = x_pad
    res = None
    for li, (w0, w1, b, g, be) in enumerate(layers[:2]):
        tx = _sc_spmm(hp, src_spmm2, dst_spmm)
        a = _tc_mma(h, w0, row(b))
        r, stats = _tc_mm(a, tx, dis16, w1[:DH], w1[DH:])
        h, hp = _tc_fin(r, stats, row(g), row(be), dis16, res=res)
        res = h

    w0, w1, b, g, be = layers[2]
    tx = _sc_spmm(hp, src_spmm2, dst_spmm)
    a = _tc_mma(h, w0, row(b))
    r3, stats3 = _tc_mm(a, tx, dis16, w1[:DH], w1[DH:])

    wr2p = jnp.pad(wr2, ((0, 0), (0, DH - 1)))
    br2p = jnp.pad(br2, (0, DH - 1)).reshape(1, DH)
    wp2p = jnp.pad(wp2, ((0, 0), (0, DH - 2)))
    bp2p = jnp.pad(bp2, (0, DH - 2)).reshape(1, DH)
    coords = _tc_heads(r3, stats3, row(g), row(be), h,
                       wr1, row(br1), row(gr), row(ber), wr2p, br2p,
                       wp1, row(bp1), row(gp), row(bep), wp2p, bp2p)
    return coords[:N, :2]
